# baseline (speedup 1.0000x reference)
import functools

import jax
import jax.numpy as jnp
from jax import lax
from jax.experimental import pallas as pl
from jax.experimental.pallas import tpu as pltpu
from jax.experimental.pallas import tpu_sc as plsc

LANES = 16
UNROLL = 4


@functools.cache
def _build(B, F, D, V):
    info = plsc.get_sparse_core_info()
    NC, NS = info.num_cores, info.num_subcores
    NW = NC * NS
    NQ = F * D
    PPW = NQ // NW
    C = V // F
    HB = B // 2
    SEGLEN = ((C + 127) // 128) * 128 + 128
    base_last = ((F - 1) * C) & ~127
    avail = V - base_last
    main_last = (avail // 128) * 128
    tail_last = avail - main_last
    assert NQ % NW == 0 and B % 2 == 0 and D % LANES == 0 and V % F == 0
    assert HB % (UNROLL * LANES) == 0

    mesh = plsc.VectorSubcoreMesh(core_axis_name="c", subcore_axis_name="s")

    @functools.partial(
        pl.kernel,
        out_type=jax.ShapeDtypeStruct((F, D, B), jnp.float32),
        mesh=mesh,
        compiler_params=pltpu.CompilerParams(use_tc_tiling_on_sc=True,
                                             needs_layout_passes=False),
        scratch_types=[
            pltpu.VMEM((1, SEGLEN), jnp.float32),
            pltpu.VMEM((1, B), jnp.int32),
            pltpu.VMEM((1, 1, HB), jnp.float32),
            pltpu.VMEM((F, D), jnp.float32),
            pltpu.SemaphoreType.DMA,
            pltpu.SemaphoreType.DMA,
        ],
    )
    def k(xt_hbm, table_hbm, bias_hbm, out_hbm, seg_v, idx_v, orow_v,
          bias_v, sem, osem):
        wid = lax.axis_index("s") * NC + lax.axis_index("c")
        q0 = wid * PPW
        pltpu.sync_copy(bias_hbm, bias_v)
        iota16 = lax.iota(jnp.int32, LANES)
        zero16 = jnp.zeros((LANES,), jnp.int32)

        def pair_body(i, fprev):
            q = q0 + i
            f = q // D
            d = q - f * D
            off = f * C
            base = pl.multiple_of(off - lax.rem(off, 128), 128)
            delta = off - base

            @pl.when(f != fprev)
            def _():
                pltpu.sync_copy(xt_hbm.at[pl.ds(f, 1)], idx_v)

                def dbody(j, c2):
                    sl = pl.ds(j * LANES, LANES)
                    idx_v[0, sl] = idx_v[0, sl] + delta
                    return c2
                lax.fori_loop(0, B // LANES, dbody, 0)

            @pl.when(f < F - 1)
            def _():
                pltpu.async_copy(
                    table_hbm.at[pl.ds(d, 1), pl.ds(base, SEGLEN)],
                    seg_v, sem)

            @pl.when(f == F - 1)
            def _():
                pltpu.async_copy(
                    table_hbm.at[pl.ds(d, 1),
                                 pl.ds(pl.multiple_of(base_last, 128),
                                       main_last)],
                    seg_v.at[:, pl.ds(0, main_last)], sem)
                pltpu.async_copy(
                    table_hbm.at[pl.ds(d, 1),
                                 pl.ds(pl.multiple_of(base_last + main_last,
                                                      128), tail_last)],
                    seg_v.at[:, pl.ds(main_last, tail_last)], sem)

            d16 = (d // LANES) * LANES
            bv = bias_v[f, pl.ds(d16, LANES)]
            bsc = jnp.sum(jnp.where(iota16 == d - d16, bv, 0.0))

            @pl.when(f < F - 1)
            def _():
                pltpu.make_async_copy(
                    table_hbm.at[pl.ds(d, 1), pl.ds(base, SEGLEN)],
                    seg_v, sem).wait()

            @pl.when(f == F - 1)
            def _():
                pltpu.make_async_copy(
                    table_hbm.at[pl.ds(d, 1), pl.ds(0, main_last)],
                    seg_v.at[:, pl.ds(0, main_last)], sem).wait()
                pltpu.make_async_copy(
                    table_hbm.at[pl.ds(d, 1), pl.ds(0, tail_last)],
                    seg_v.at[:, pl.ds(main_last, tail_last)], sem).wait()

            for h in range(2):
                hb = h * HB

                def gbody(j, c2, hb=hb, h=h):
                    for u in range(UNROLL):
                        p = j * UNROLL * LANES + u * LANES
                        iv = idx_v[0, pl.ds(hb + p, LANES)]
                        g = plsc.load_gather(seg_v, [zero16, iv])
                        orow_v[0, 0, pl.ds(p, LANES)] = g + bsc
                    return c2
                lax.fori_loop(0, HB // (UNROLL * LANES), gbody, 0)
                pltpu.sync_copy(orow_v, out_hbm.at[pl.ds(f, 1), pl.ds(d, 1), pl.ds(hb, HB)])
            return f
        lax.fori_loop(0, PPW, pair_body, -1)

    return k


def kernel(x, offsets, table, bias):
    B, F = x.shape
    V, D = table.shape
    k = _build(B, F, D, V)
    out3 = k(x.T, table.T, bias)
    return jnp.transpose(out3, (2, 0, 1))
